# Initial kernel scaffold; baseline (speedup 1.0000x reference)
import functools

import jax
import jax.numpy as jnp
from jax import lax
from jax.experimental import pallas as pl
from jax.experimental.pallas import tpu as pltpu
from jax.experimental.pallas import tpu_sc as plsc

_N = 10000
_D = 128
_E = 320000
_DE = 16
_EPS = 1e-05

_NC = 2
_NS = 16
_NW = _NC * _NS
_EPW = _E // _NW
_C = 80
_NCHUNK = _EPW // _C
_RPT = _N // _NS


def _edge_mm_body(a_ref, w_ref, o_ref):
    o_ref[...] = lax.dot_general(
        a_ref[...], w_ref[...], (((1,), (1,)), ((), ())),
        preferred_element_type=jnp.float32)


_BE = 2000


def _edge_mm(edge_attr, W_edge):
    return pl.pallas_call(
        _edge_mm_body,
        grid=(_E // _BE,),
        in_specs=[pl.BlockSpec((_BE, _DE), lambda i: (i, 0)),
                  pl.BlockSpec((_D, _DE), lambda i: (0, 0))],
        out_specs=pl.BlockSpec((_BE, _D), lambda i: (i, 0)),
        out_shape=jax.ShapeDtypeStruct((_E, _D), jnp.float32),
    )(edge_attr, W_edge)


def _sc_agg(x, src, dst, e):
    mesh = plsc.VectorSubcoreMesh(core_axis_name="c", subcore_axis_name="s")

    @functools.partial(
        pl.kernel,
        mesh=mesh,
        out_type=jax.ShapeDtypeStruct((_NC * _N, _D), jnp.float32),
        scratch_types=[
            pltpu.VMEM((_C,), jnp.int32),
            pltpu.VMEM((1, _C), jnp.int32),
            pltpu.VMEM((_C, _D), jnp.float32),
            pltpu.VMEM((_C, _D), jnp.float32),
            pltpu.VMEM_SHARED((_N, _D), jnp.float32),
            pltpu.SemaphoreType.DMA,
            pltpu.SemaphoreType.DMA,
        ],
    )
    def agg_kernel(x_hbm, src_hbm, dst_hbm, e_hbm, out_hbm,
                   idxs_v, idxd_v, xrows_v, erows_v, agg_sh, sem_g, sem_e):
        cid = lax.axis_index("c")
        sid = lax.axis_index("s")
        wid = sid * _NC + cid

        @pl.loop(0, _C)
        def _(r):
            for g in range(_D // 16):
                erows_v[r, pl.ds(g * 16, 16)] = jnp.zeros((16,), jnp.float32)

        zbase = sid * _RPT
        for j in range(_RPT // _C):
            pltpu.sync_copy(erows_v, agg_sh.at[pl.ds(zbase + j * _C, _C)])
        _ztail = _RPT % _C
        if _ztail:
            pltpu.sync_copy(erows_v.at[pl.ds(0, _ztail)],
                            agg_sh.at[pl.ds(zbase + (_RPT // _C) * _C, _ztail)])
        plsc.subcore_barrier()

        @pl.loop(0, _NCHUNK)
        def _(i):
            base = wid * _EPW + i * _C
            pltpu.sync_copy(src_hbm.at[pl.ds(base, _C)], idxs_v)
            pltpu.sync_copy(dst_hbm.at[pl.ds(base, _C)], idxd_v.at[0])
            ce = pltpu.async_copy(e_hbm.at[pl.ds(base, _C)], erows_v, sem_e)
            cg = pltpu.async_copy(x_hbm.at[idxs_v], xrows_v, sem_g)
            ce.wait()
            cg.wait()

            @pl.loop(0, _C)
            def _(r):
                for g in range(_D // 16):
                    sl = pl.ds(g * 16, 16)
                    xrows_v[r, sl] = jnp.maximum(
                        xrows_v[r, sl] + erows_v[r, sl], 0.0)

            pltpu.sync_copy(xrows_v, agg_sh.at[idxd_v.at[0]], add=True)

        plsc.subcore_barrier()

        row0 = cid * _N + sid * _RPT
        pltpu.sync_copy(agg_sh.at[pl.ds(sid * _RPT, _RPT)],
                        out_hbm.at[pl.ds(row0, _RPT)])

    return agg_kernel(x, src, dst, e)


def _final_body(x_ref, p0_ref, p1_ref, w_ref, b_ref, o_ref):
    h = x_ref[...] * (1.0 + _EPS) + p0_ref[...] + p1_ref[...]
    h = lax.dot_general(h, w_ref[...], (((1,), (1,)), ((), ())),
                        preferred_element_type=jnp.float32)
    o_ref[...] = jnp.maximum(h + b_ref[...], 0.0)


_BN = 2000


def _final(x, p0, p1, W_nn, b_nn2):
    return pl.pallas_call(
        _final_body,
        grid=(_N // _BN,),
        in_specs=[pl.BlockSpec((_BN, _D), lambda i: (i, 0)),
                  pl.BlockSpec((_BN, _D), lambda i: (i, 0)),
                  pl.BlockSpec((_BN, _D), lambda i: (i, 0)),
                  pl.BlockSpec((_D, _D), lambda i: (0, 0)),
                  pl.BlockSpec((1, _D), lambda i: (0, 0))],
        out_specs=pl.BlockSpec((_BN, _D), lambda i: (i, 0)),
        out_shape=jax.ShapeDtypeStruct((_N, _D), jnp.float32),
    )(x, p0, p1, W_nn, b_nn2)


def kernel(x, edge_index, edge_attr, W_edge, W_nn, b_nn):
    src = edge_index[0]
    dst = edge_index[1]
    e = _edge_mm(edge_attr, W_edge)
    parts = _sc_agg(x, src, dst, e)
    return _final(x, parts[:_N], parts[_N:], W_nn, b_nn.reshape(1, _D))


# speedup vs baseline: 2.6621x; 2.6621x over previous
import functools

import jax
import jax.numpy as jnp
from jax import lax
from jax.experimental import pallas as pl
from jax.experimental.pallas import tpu as pltpu
from jax.experimental.pallas import tpu_sc as plsc

_N = 10000
_D = 128
_E = 320000
_DE = 16
_EPS = 1e-05

_NC = 2
_NS = 16
_NW = _NC * _NS
_EPW = _E // _NW
_C = 80
_NCHUNK = _EPW // _C
_RPT = 624
_RTAIL = _N - _NS * _RPT


def _edge_mm_body(a_ref, w_ref, o_ref):
    o_ref[...] = lax.dot_general(
        a_ref[...], w_ref[...], (((1,), (1,)), ((), ())),
        preferred_element_type=jnp.float32)


_BE = 2000


def _edge_mm(edge_attr, W_edge):
    return pl.pallas_call(
        _edge_mm_body,
        grid=(_E // _BE,),
        in_specs=[pl.BlockSpec((_BE, _DE), lambda i: (i, 0)),
                  pl.BlockSpec((_D, _DE), lambda i: (0, 0))],
        out_specs=pl.BlockSpec((_BE, _D), lambda i: (i, 0)),
        out_shape=jax.ShapeDtypeStruct((_E, _D), jnp.float32),
    )(edge_attr, W_edge)


def _sc_agg(x, src, dst, e):
    mesh = plsc.VectorSubcoreMesh(core_axis_name="c", subcore_axis_name="s")

    @functools.partial(
        pl.kernel,
        mesh=mesh,
        out_type=jax.ShapeDtypeStruct((_NC * _N, _D), jnp.float32),
        scratch_types=[
            pltpu.VMEM((_C,), jnp.int32),
            pltpu.VMEM((1, _C), jnp.int32),
            pltpu.VMEM((_C, _D), jnp.float32),
            pltpu.VMEM((_C, _D), jnp.float32),
            pltpu.VMEM_SHARED((_N, _D), jnp.float32),
            pltpu.SemaphoreType.DMA,
            pltpu.SemaphoreType.DMA,
        ],
    )
    def agg_kernel(x_hbm, src_hbm, dst_hbm, e_hbm, out_hbm,
                   idxs_v, idxd_v, xrows_v, erows_v, agg_sh, sem_g, sem_e):
        cid = lax.axis_index("c")
        sid = lax.axis_index("s")
        wid = sid * _NC + cid

        @pl.loop(0, _C)
        def _(r):
            for g in range(_D // 16):
                erows_v[r, pl.ds(g * 16, 16)] = jnp.zeros((16,), jnp.float32)

        zbase = sid * _RPT
        for j in range(_RPT // _C):
            pltpu.sync_copy(erows_v, agg_sh.at[pl.ds(zbase + j * _C, _C)])
        _ztail = _RPT % _C
        if _ztail:
            pltpu.sync_copy(erows_v.at[pl.ds(0, _ztail)],
                            agg_sh.at[pl.ds(zbase + (_RPT // _C) * _C, _ztail)])

        @pl.when(sid == _NS - 1)
        def _():
            pltpu.sync_copy(erows_v.at[pl.ds(0, _RTAIL)],
                            agg_sh.at[pl.ds(_NS * _RPT, _RTAIL)])

        plsc.subcore_barrier()

        @pl.loop(0, _NCHUNK)
        def _(i):
            base = wid * _EPW + i * _C
            pltpu.sync_copy(src_hbm.at[pl.ds(base, _C)], idxs_v)
            pltpu.sync_copy(dst_hbm.at[pl.ds(base, _C)], idxd_v.at[0])
            ce = pltpu.async_copy(e_hbm.at[pl.ds(base, _C)], erows_v, sem_e)
            cg = pltpu.async_copy(x_hbm.at[idxs_v], xrows_v, sem_g)
            ce.wait()
            cg.wait()

            @pl.loop(0, _C)
            def _(r):
                for g in range(_D // 16):
                    sl = pl.ds(g * 16, 16)
                    xrows_v[r, sl] = jnp.maximum(
                        xrows_v[r, sl] + erows_v[r, sl], 0.0)

            pltpu.sync_copy(xrows_v, agg_sh.at[idxd_v.at[0]], add=True)

        plsc.subcore_barrier()

        row0 = cid * _N + sid * _RPT
        pltpu.sync_copy(agg_sh.at[pl.ds(sid * _RPT, _RPT)],
                        out_hbm.at[pl.ds(row0, _RPT)])

        @pl.when(sid == _NS - 1)
        def _():
            pltpu.sync_copy(agg_sh.at[pl.ds(_NS * _RPT, _RTAIL)],
                            out_hbm.at[pl.ds(cid * _N + _NS * _RPT, _RTAIL)])

    return agg_kernel(x, src, dst, e)


def _final_body(x_ref, p0_ref, p1_ref, w_ref, b_ref, o_ref):
    h = x_ref[...] * (1.0 + _EPS) + p0_ref[...] + p1_ref[...]
    h = lax.dot_general(h, w_ref[...], (((1,), (1,)), ((), ())),
                        preferred_element_type=jnp.float32)
    o_ref[...] = jnp.maximum(h + b_ref[...], 0.0)


_BN = 2000


def _final(x, p0, p1, W_nn, b_nn2):
    return pl.pallas_call(
        _final_body,
        grid=(_N // _BN,),
        in_specs=[pl.BlockSpec((_BN, _D), lambda i: (i, 0)),
                  pl.BlockSpec((_BN, _D), lambda i: (i, 0)),
                  pl.BlockSpec((_BN, _D), lambda i: (i, 0)),
                  pl.BlockSpec((_D, _D), lambda i: (0, 0)),
                  pl.BlockSpec((1, _D), lambda i: (0, 0))],
        out_specs=pl.BlockSpec((_BN, _D), lambda i: (i, 0)),
        out_shape=jax.ShapeDtypeStruct((_N, _D), jnp.float32),
    )(x, p0, p1, W_nn, b_nn2)


def kernel(x, edge_index, edge_attr, W_edge, W_nn, b_nn):
    src = edge_index[0]
    dst = edge_index[1]
    e = _edge_mm(edge_attr, W_edge)
    parts = _sc_agg(x, src, dst, e)
    return _final(x, parts[:_N], parts[_N:], W_nn, b_nn.reshape(1, _D))


# speedup vs baseline: 3.2053x; 1.2041x over previous
import functools

import jax
import jax.numpy as jnp
from jax import lax
from jax.experimental import pallas as pl
from jax.experimental.pallas import tpu as pltpu
from jax.experimental.pallas import tpu_sc as plsc

_N = 10000
_D = 128
_E = 320000
_DE = 16
_EPS = 1e-05

_NC = 2
_NS = 16
_NW = _NC * _NS
_EPW = _E // _NW
_C = 80
_NCHUNK = _EPW // _C
_RPT = 624
_RTAIL = _N - _NS * _RPT


def _edge_mm_body(a_ref, w_ref, o_ref):
    o_ref[...] = lax.dot_general(
        a_ref[...], w_ref[...], (((1,), (1,)), ((), ())),
        preferred_element_type=jnp.float32)


_BE = 2000


def _edge_mm(edge_attr, W_edge):
    return pl.pallas_call(
        _edge_mm_body,
        grid=(_E // _BE,),
        in_specs=[pl.BlockSpec((_BE, _DE), lambda i: (i, 0)),
                  pl.BlockSpec((_D, _DE), lambda i: (0, 0))],
        out_specs=pl.BlockSpec((_BE, _D), lambda i: (i, 0)),
        out_shape=jax.ShapeDtypeStruct((_E, _D), jnp.float32),
    )(edge_attr, W_edge)


def _sc_agg(x, src, dst, e):
    mesh = plsc.VectorSubcoreMesh(core_axis_name="c", subcore_axis_name="s")

    @functools.partial(
        pl.kernel,
        mesh=mesh,
        out_type=jax.ShapeDtypeStruct((_NC * _N, _D), jnp.float32),
        scratch_types=[
            pltpu.VMEM((2, _C), jnp.int32),
            pltpu.VMEM((2, _C), jnp.int32),
            pltpu.VMEM((2, _C, _D), jnp.float32),
            pltpu.VMEM((2, _C, _D), jnp.float32),
            pltpu.VMEM_SHARED((_N, _D), jnp.float32),
            pltpu.SemaphoreType.DMA,
            pltpu.SemaphoreType.DMA,
        ],
    )
    def agg_kernel(x_hbm, src_hbm, dst_hbm, e_hbm, out_hbm,
                   idxs_v, idxd_v, xrows_v, erows_v, agg_sh, sem_g, sem_e):
        cid = lax.axis_index("c")
        sid = lax.axis_index("s")
        wid = sid * _NC + cid

        zrows = erows_v.at[0]

        @pl.loop(0, _C)
        def _(r):
            for g in range(_D // 16):
                zrows[r, pl.ds(g * 16, 16)] = jnp.zeros((16,), jnp.float32)

        zbase = sid * _RPT
        for j in range(_RPT // _C):
            pltpu.sync_copy(zrows, agg_sh.at[pl.ds(zbase + j * _C, _C)])
        _ztail = _RPT % _C
        if _ztail:
            pltpu.sync_copy(zrows.at[pl.ds(0, _ztail)],
                            agg_sh.at[pl.ds(zbase + (_RPT // _C) * _C, _ztail)])

        @pl.when(sid == _NS - 1)
        def _():
            pltpu.sync_copy(zrows.at[pl.ds(0, _RTAIL)],
                            agg_sh.at[pl.ds(_NS * _RPT, _RTAIL)])

        plsc.subcore_barrier()

        def issue(g, slot):
            base = wid * _EPW + g * _C
            pltpu.sync_copy(src_hbm.at[pl.ds(base, _C)], idxs_v.at[slot])
            pltpu.sync_copy(dst_hbm.at[pl.ds(base, _C)], idxd_v.at[slot])
            pltpu.async_copy(e_hbm.at[pl.ds(base, _C)], erows_v.at[slot],
                             sem_e)
            pltpu.async_copy(x_hbm.at[idxs_v.at[slot]], xrows_v.at[slot],
                             sem_g)

        def wait_slot(slot):
            pltpu.make_async_copy(e_hbm.at[pl.ds(0, _C)], erows_v.at[slot],
                                  sem_e).wait()
            pltpu.make_async_copy(x_hbm.at[idxs_v.at[slot]],
                                  xrows_v.at[slot], sem_g).wait()

        def compute_scatter(slot):
            xr = xrows_v.at[slot]
            er = erows_v.at[slot]

            @pl.loop(0, _C)
            def _(r):
                for g in range(_D // 16):
                    sl = pl.ds(g * 16, 16)
                    xr[r, sl] = jnp.maximum(xr[r, sl] + er[r, sl], 0.0)

            pltpu.sync_copy(xr, agg_sh.at[idxd_v.at[slot]], add=True)

        issue(0, 0)

        @pl.loop(0, (_NCHUNK - 1) // 2)
        def _(i):
            g0 = 2 * i
            wait_slot(0)
            issue(g0 + 1, 1)
            compute_scatter(0)
            wait_slot(1)
            issue(g0 + 2, 0)
            compute_scatter(1)

        wait_slot(0)
        compute_scatter(0)

        plsc.subcore_barrier()

        row0 = cid * _N + sid * _RPT
        pltpu.sync_copy(agg_sh.at[pl.ds(sid * _RPT, _RPT)],
                        out_hbm.at[pl.ds(row0, _RPT)])

        @pl.when(sid == _NS - 1)
        def _():
            pltpu.sync_copy(agg_sh.at[pl.ds(_NS * _RPT, _RTAIL)],
                            out_hbm.at[pl.ds(cid * _N + _NS * _RPT, _RTAIL)])

    return agg_kernel(x, src, dst, e)


def _final_body(x_ref, p0_ref, p1_ref, w_ref, b_ref, o_ref):
    h = x_ref[...] * (1.0 + _EPS) + p0_ref[...] + p1_ref[...]
    h = lax.dot_general(h, w_ref[...], (((1,), (1,)), ((), ())),
                        preferred_element_type=jnp.float32)
    o_ref[...] = jnp.maximum(h + b_ref[...], 0.0)


_BN = 2000


def _final(x, p0, p1, W_nn, b_nn2):
    return pl.pallas_call(
        _final_body,
        grid=(_N // _BN,),
        in_specs=[pl.BlockSpec((_BN, _D), lambda i: (i, 0)),
                  pl.BlockSpec((_BN, _D), lambda i: (i, 0)),
                  pl.BlockSpec((_BN, _D), lambda i: (i, 0)),
                  pl.BlockSpec((_D, _D), lambda i: (0, 0)),
                  pl.BlockSpec((1, _D), lambda i: (0, 0))],
        out_specs=pl.BlockSpec((_BN, _D), lambda i: (i, 0)),
        out_shape=jax.ShapeDtypeStruct((_N, _D), jnp.float32),
    )(x, p0, p1, W_nn, b_nn2)


def kernel(x, edge_index, edge_attr, W_edge, W_nn, b_nn):
    src = edge_index[0]
    dst = edge_index[1]
    e = _edge_mm(edge_attr, W_edge)
    parts = _sc_agg(x, src, dst, e)
    return _final(x, parts[:_N], parts[_N:], W_nn, b_nn.reshape(1, _D))
